# Initial kernel scaffold; baseline (speedup 1.0000x reference)
import jax
import jax.numpy as jnp
from jax.experimental import pallas as pl


def kernel(x_in):
    raise NotImplementedError("write your pallas kernel here")


# speedup vs baseline: 3.8762x; 3.8762x over previous
import functools

import numpy as np
import jax
import jax.numpy as jnp
from jax.experimental import pallas as pl
from jax.experimental.pallas import tpu as pltpu

_K = 5
_BW = 10


def _np_box(a, k):
    r = k // 2
    H, W = a.shape
    p = np.zeros((H + 2 * r, W + 2 * r), a.dtype)
    p[r:r + H, r:r + W] = a
    out = np.zeros((H, W), a.dtype)
    for di in range(k):
        for dj in range(k):
            out += p[di:di + H, dj:dj + W]
    return out


@functools.lru_cache()
def _consts(H, W):
    ii = np.arange(H)
    jj = np.arange(W)
    rb = (ii < _BW) | (ii >= H - _BW)
    cb = (jj < _BW) | (jj >= W - _BW)
    mb = rb[:, None] | cb[None, :]
    mbf = mb.astype(np.float32)
    cnt_b = np.maximum(_np_box(mbf, _K), 1.0).astype(np.float32)
    cnt_i = np.maximum(_np_box((1.0 - mbf).astype(np.float32), _K), 1.0)
    wb = np.where(mb, 1.0 / cnt_b, -1.0 / cnt_i).astype(np.float32)
    wi = np.where(mb, 0.0, 1.0 / cnt_i).astype(np.float32)
    return np.stack([wb, wi])


def _shift_rows(a, s):
    H, W = a.shape
    z = jnp.zeros((abs(s), W), a.dtype)
    if s > 0:
        return jnp.concatenate([z, a[: H - s, :]], axis=0)
    return jnp.concatenate([a[-s:, :], z], axis=0)


def _shift_cols(a, s):
    H, W = a.shape
    z = jnp.zeros((H, abs(s)), a.dtype)
    if s > 0:
        return jnp.concatenate([z, a[:, : W - s]], axis=1)
    return jnp.concatenate([a[:, -s:], z], axis=1)


def _tap5(a, shift):
    return shift(a, 2) + shift(a, 1) + a + shift(a, -1) + shift(a, -2)


def _box5(a):
    return _tap5(_tap5(a, _shift_rows), _shift_cols)


def _body(x_ref, c_ref, o_ref):
    X = x_ref[0]
    wb = c_ref[0]
    S = _box5(X)
    Xb = jnp.where(wb > 0, X, 0.0)
    B = _box5(Xb)
    o_ref[0] = B * wb + S * c_ref[1]


def kernel(x_in):
    N1, N2, H, W = x_in.shape
    n = N1 * N2
    x = x_in.reshape(n, H, W)
    consts = jnp.asarray(_consts(H, W))
    out = pl.pallas_call(
        _body,
        out_shape=jax.ShapeDtypeStruct((n, H, W), x_in.dtype),
        grid=(n,),
        in_specs=[
            pl.BlockSpec((1, H, W), lambda i: (i, 0, 0)),
            pl.BlockSpec((2, H, W), lambda i: (0, 0, 0)),
        ],
        out_specs=pl.BlockSpec((1, H, W), lambda i: (i, 0, 0)),
        compiler_params=pltpu.CompilerParams(
            dimension_semantics=("arbitrary",),
            vmem_limit_bytes=60 * 1024 * 1024,
        ),
        name="segmented_smoothing",
    )(x, consts)
    return out.reshape(N1, N2, H, W)


# speedup vs baseline: 9.0029x; 2.3226x over previous
import functools

import numpy as np
import jax
import jax.numpy as jnp
from jax.experimental import pallas as pl
from jax.experimental.pallas import tpu as pltpu

_K = 5
_BW = 10
_SW = 16
_OW = 12


def _np_box(a, k):
    r = k // 2
    H, W = a.shape
    p = np.zeros((H + 2 * r, W + 2 * r), a.dtype)
    p[r:r + H, r:r + W] = a
    out = np.zeros((H, W), a.dtype)
    for di in range(k):
        for dj in range(k):
            out += p[di:di + H, dj:dj + W]
    return out


@functools.lru_cache()
def _consts(H, W):
    ii = np.arange(H)
    jj = np.arange(W)
    rb = (ii < _BW) | (ii >= H - _BW)
    cb = (jj < _BW) | (jj >= W - _BW)
    mb = rb[:, None] | cb[None, :]
    mbf = mb.astype(np.float32)
    cnt_b = np.maximum(_np_box(mbf, _K), 1.0).astype(np.float32)
    cnt_i = np.maximum(_np_box((1.0 - mbf).astype(np.float32), _K), 1.0)
    wb = np.where(mb, 1.0 / cnt_b, -1.0 / cnt_i).astype(np.float32)
    wi = np.where(mb, 0.0, 1.0 / cnt_i).astype(np.float32)

    top = np.stack([wb[:_SW], wi[:_SW]])
    bot = np.stack([wb[H - _SW:], wi[H - _SW:]])
    colp = np.stack([
        np.concatenate([wb[:, :_SW], wb[:, W - _SW:]], axis=1),
        np.concatenate([wi[:, :_SW], wi[:, W - _SW:]], axis=1),
    ])
    return top, bot, colp


def _shift_rows(a, s):
    H, W = a.shape
    z = jnp.zeros((abs(s), W), a.dtype)
    if s > 0:
        return jnp.concatenate([z, a[: H - s, :]], axis=0)
    return jnp.concatenate([a[-s:, :], z], axis=0)


def _shift_cols(a, s):
    H, W = a.shape
    z = jnp.zeros((H, abs(s)), a.dtype)
    if s > 0:
        return jnp.concatenate([z, a[:, : W - s]], axis=1)
    return jnp.concatenate([a[:, -s:], z], axis=1)


def _tap5(a, shift):
    return shift(a, 2) + shift(a, 1) + a + shift(a, -1) + shift(a, -2)


def _tap5_fast(a, shift):
    q = a + shift(a, -1)
    return shift(q, 2) + q + shift(a, -2)


def _box5(a):
    return _tap5(_tap5(a, _shift_rows), _shift_cols)


def _box5_fast(a):
    return _tap5_fast(_tap5_fast(a, _shift_rows), _shift_cols)


def _strip(Xs, wb, wi):
    S = _box5(Xs)
    B = _box5(jnp.where(wb > 0, Xs, 0.0))
    return B * wb + S * wi


def _body(x_ref, ct_ref, cb_ref, cc_ref, o_ref):
    X = x_ref[0]
    H, W = X.shape

    o_ref[0] = _box5_fast(X) * jnp.float32(1.0 / (_K * _K))

    top = _strip(X[:_SW], ct_ref[0], ct_ref[1])
    o_ref[0, :_OW, :] = top[:_OW]
    bot = _strip(X[H - _SW:], cb_ref[0], cb_ref[1])
    o_ref[0, H - _OW:, :] = bot[_SW - _OW:]

    P = jnp.concatenate([X[:, :_SW], X[:, W - _SW:]], axis=1)
    colr = _strip(P, cc_ref[0], cc_ref[1])
    o_ref[0, :, :_OW] = colr[:, :_OW]
    o_ref[0, :, W - _OW:] = colr[:, 2 * _SW - _OW:]


def kernel(x_in):
    N1, N2, H, W = x_in.shape
    n = N1 * N2
    x = x_in.reshape(n, H, W)
    top, bot, colp = _consts(H, W)
    ct = jnp.asarray(top)
    cb = jnp.asarray(bot)
    cc = jnp.asarray(colp)
    out = pl.pallas_call(
        _body,
        out_shape=jax.ShapeDtypeStruct((n, H, W), x_in.dtype),
        grid=(n,),
        in_specs=[
            pl.BlockSpec((1, H, W), lambda i: (i, 0, 0)),
            pl.BlockSpec((2, _SW, W), lambda i: (0, 0, 0)),
            pl.BlockSpec((2, _SW, W), lambda i: (0, 0, 0)),
            pl.BlockSpec((2, H, 2 * _SW), lambda i: (0, 0, 0)),
        ],
        out_specs=pl.BlockSpec((1, H, W), lambda i: (i, 0, 0)),
        compiler_params=pltpu.CompilerParams(
            dimension_semantics=("arbitrary",),
            vmem_limit_bytes=60 * 1024 * 1024,
        ),
        name="segmented_smoothing",
    )(x, ct, cb, cc)
    return out.reshape(N1, N2, H, W)


# speedup vs baseline: 9.0249x; 1.0024x over previous
import functools

import numpy as np
import jax
import jax.numpy as jnp
from jax.experimental import pallas as pl
from jax.experimental.pallas import tpu as pltpu

_K = 5
_BW = 10
_SW = 16
_OW = 12


def _np_box(a, k):
    r = k // 2
    H, W = a.shape
    p = np.zeros((H + 2 * r, W + 2 * r), a.dtype)
    p[r:r + H, r:r + W] = a
    out = np.zeros((H, W), a.dtype)
    for di in range(k):
        for dj in range(k):
            out += p[di:di + H, dj:dj + W]
    return out


@functools.lru_cache()
def _consts(H, W):
    ii = np.arange(H)
    jj = np.arange(W)
    rb = (ii < _BW) | (ii >= H - _BW)
    cb = (jj < _BW) | (jj >= W - _BW)
    mb = rb[:, None] | cb[None, :]
    mbf = mb.astype(np.float32)
    cnt_b = np.maximum(_np_box(mbf, _K), 1.0).astype(np.float32)
    cnt_i = np.maximum(_np_box((1.0 - mbf).astype(np.float32), _K), 1.0)
    wb = np.where(mb, 1.0 / cnt_b, -1.0 / cnt_i).astype(np.float32)
    wi = np.where(mb, 0.0, 1.0 / cnt_i).astype(np.float32)

    top = np.stack([wb[:_SW], wi[:_SW]])
    bot = np.stack([wb[H - _SW:], wi[H - _SW:]])
    colp = np.stack([
        np.concatenate([wb[:, :_SW], wb[:, W - _SW:]], axis=1),
        np.concatenate([wi[:, :_SW], wi[:, W - _SW:]], axis=1),
    ])
    return top, bot, colp


def _shift_rows(a, s):
    H, W = a.shape
    z = jnp.zeros((abs(s), W), a.dtype)
    if s > 0:
        return jnp.concatenate([z, a[: H - s, :]], axis=0)
    return jnp.concatenate([a[-s:, :], z], axis=0)


def _shift_cols(a, s):
    H, W = a.shape
    z = jnp.zeros((H, abs(s)), a.dtype)
    if s > 0:
        return jnp.concatenate([z, a[:, : W - s]], axis=1)
    return jnp.concatenate([a[:, -s:], z], axis=1)


def _tap5(a, shift):
    return shift(a, 2) + shift(a, 1) + a + shift(a, -1) + shift(a, -2)


def _tap5_fast(a, shift):
    q = a + shift(a, -1)
    return shift(q, 2) + q + shift(a, -2)


def _box5(a):
    return _tap5(_tap5(a, _shift_rows), _shift_cols)


def _box5_fast(a):
    return _tap5_fast(_tap5_fast(a, _shift_rows), _shift_cols)


def _strip(Xs, wb, wi):
    S = _box5(Xs)
    B = _box5(jnp.where(wb > 0, Xs, 0.0))
    return B * wb + S * wi


def _body(x_ref, ct_ref, cb_ref, cc_ref, o_ref):
    nimg = x_ref.shape[0]
    for b in range(nimg):
        X = x_ref[b]
        H, W = X.shape

        o_ref[b] = _box5_fast(X) * jnp.float32(1.0 / (_K * _K))

        top = _strip(X[:_SW], ct_ref[0], ct_ref[1])
        o_ref[b, :_OW, :] = top[:_OW]
        bot = _strip(X[H - _SW:], cb_ref[0], cb_ref[1])
        o_ref[b, H - _OW:, :] = bot[_SW - _OW:]

        P = jnp.concatenate([X[:, :_SW], X[:, W - _SW:]], axis=1)
        colr = _strip(P, cc_ref[0], cc_ref[1])
        o_ref[b, :, :_OW] = colr[:, :_OW]
        o_ref[b, :, W - _OW:] = colr[:, 2 * _SW - _OW:]


def kernel(x_in):
    N1, N2, H, W = x_in.shape
    n = N1 * N2
    x = x_in.reshape(n, H, W)
    top, bot, colp = _consts(H, W)
    ct = jnp.asarray(top)
    cb = jnp.asarray(bot)
    cc = jnp.asarray(colp)
    nimg = 2 if n % 2 == 0 else 1
    out = pl.pallas_call(
        _body,
        out_shape=jax.ShapeDtypeStruct((n, H, W), x_in.dtype),
        grid=(n // nimg,),
        in_specs=[
            pl.BlockSpec((nimg, H, W), lambda i: (i, 0, 0)),
            pl.BlockSpec((2, _SW, W), lambda i: (0, 0, 0)),
            pl.BlockSpec((2, _SW, W), lambda i: (0, 0, 0)),
            pl.BlockSpec((2, H, 2 * _SW), lambda i: (0, 0, 0)),
        ],
        out_specs=pl.BlockSpec((nimg, H, W), lambda i: (i, 0, 0)),
        compiler_params=pltpu.CompilerParams(
            dimension_semantics=("arbitrary",),
            vmem_limit_bytes=60 * 1024 * 1024,
        ),
        name="segmented_smoothing",
    )(x, ct, cb, cc)
    return out.reshape(N1, N2, H, W)
